# Initial kernel scaffold; baseline (speedup 1.0000x reference)
import jax
import jax.numpy as jnp
from jax.experimental import pallas as pl


def kernel(p1, x1, params):
    raise NotImplementedError("write your pallas kernel here")


# speedup vs baseline: 21.6293x; 21.6293x over previous
import functools

import jax
import jax.numpy as jnp
from jax import lax
from jax.experimental import pallas as pl
from jax.experimental.pallas import tpu as pltpu
from jax.experimental.pallas import tpu_sc as plsc

F32 = jnp.float32


def _relu(x):
    return jnp.maximum(x, 0.0)


def _pad128(n):
    return (n + 127) // 128 * 128


def _dot(a, b):
    return jnp.dot(a, b, preferred_element_type=F32)


def _row8(v):
    return jnp.broadcast_to(v[None, :], (8, v.shape[0]))


def _sc_gather(table, idx):
    R, D = table.shape
    G = idx.shape[0]
    assert G % 128 == 0 and D % 128 == 0
    nch = G // 128
    idx2 = idx.reshape(nch, 128)
    NW = 32
    nfull, rem = divmod(nch, NW)
    mesh = plsc.VectorSubcoreMesh(core_axis_name="c", subcore_axis_name="s")

    @functools.partial(
        pl.kernel,
        out_type=jax.ShapeDtypeStruct((G, D), F32),
        mesh=mesh,
        scratch_types=[
            pltpu.VMEM((128,), jnp.int32),
            pltpu.VMEM((128, D), F32),
            pltpu.SemaphoreType.DMA,
        ],
    )
    def gk(tab_hbm, idx_hbm, out_hbm, idx_v, rows_v, sem):
        wid = lax.axis_index("s") * 2 + lax.axis_index("c")
        my = nfull + jnp.where(wid < rem, 1, 0)

        def body(i, carry):
            ci = i * NW + wid
            pltpu.sync_copy(idx_hbm.at[ci], idx_v)
            pltpu.async_copy(tab_hbm.at[idx_v], rows_v, sem).wait()
            pltpu.sync_copy(rows_v, out_hbm.at[pl.ds(ci * 128, 128)])
            return carry

        lax.fori_loop(0, my, body, 0)

    return gk(table, idx2)


def _flat_idx(idx, nrows):
    B = idx.shape[0]
    perm = jnp.transpose(idx, (0, 2, 1))
    offs = (jnp.arange(B, dtype=jnp.int32) * nrows)[:, None, None]
    return (perm + offs).reshape(-1).astype(jnp.int32)


def _embed(x1t, w, g, b):
    B, N, Kp = x1t.shape
    C = w.shape[1]

    def body(x_ref, w_ref, g_ref, b_ref, o_ref):
        h = _dot(x_ref[0], w_ref[...])
        mu = jnp.mean(h)
        var = jnp.mean((h - mu) ** 2)
        hn = (h - mu) / jnp.sqrt(var + 1e-5) * g_ref[0:1, :] + b_ref[0:1, :]
        o_ref[0] = _relu(hn)

    return pl.pallas_call(
        body,
        grid=(B,),
        in_specs=[
            pl.BlockSpec((1, N, Kp), lambda i: (i, 0, 0)),
            pl.BlockSpec((Kp, C), lambda i: (0, 0)),
            pl.BlockSpec((8, C), lambda i: (0, 0)),
            pl.BlockSpec((8, C), lambda i: (0, 0)),
        ],
        out_specs=pl.BlockSpec((1, N, C), lambda i: (i, 0, 0)),
        out_shape=jax.ShapeDtypeStruct((B, N, C), F32),
    )(x1t, w, _row8(g), _row8(b))


def _knn(pq, prT, k):
    B, M, _ = pq.shape
    N = prT.shape[2]
    MT = min(M, max(8, (1 << 21) // N))

    def body(q_ref, r_ref, o_ref):
        q = q_ref[0]
        rT = r_ref[0]
        cross = _dot(q, rT)
        sqq = jnp.sum(q * q, axis=1, keepdims=True)
        sqr = jnp.sum(rT * rT, axis=0, keepdims=True)
        d = sqq - 2.0 * cross + sqr
        ii = lax.broadcasted_iota(jnp.int32, (MT, N), 1)
        cols = []
        for _ in range(k):
            m = jnp.min(d, axis=1, keepdims=True)
            cand = jnp.where(d == m, ii, N)
            sel = jnp.min(cand, axis=1, keepdims=True)
            cols.append(sel)
            d = jnp.where(ii == sel, 1e30, d)
        o_ref[0] = jnp.concatenate(cols, axis=1)

    return pl.pallas_call(
        body,
        grid=(B, M // MT),
        in_specs=[
            pl.BlockSpec((1, MT, 16), lambda i, j: (i, j, 0)),
            pl.BlockSpec((1, 16, N), lambda i, j: (i, 0, 0)),
        ],
        out_specs=pl.BlockSpec((1, MT, k), lambda i, j: (i, j, 0)),
        out_shape=jax.ShapeDtypeStruct((B, M, k), jnp.int32),
    )(pq, prT)


def _proj(x, pp, bp):
    B, N, C = x.shape
    TD = _pad128(2 * C + 16)

    def body(x_ref, p_ref, l1, wq, wk, wv, q_ref, t_ref):
        h = _dot(x_ref[0], l1[...])
        q_ref[0] = _dot(h, wq[...])
        t_ref[0] = jnp.concatenate(
            [_dot(h, wk[...]), _dot(h, wv[...]), p_ref[0],
             jnp.zeros((N, TD - 2 * C - 16), F32)], axis=1)

    return pl.pallas_call(
        body,
        grid=(B,),
        in_specs=[
            pl.BlockSpec((1, N, C), lambda i: (i, 0, 0)),
            pl.BlockSpec((1, N, 16), lambda i: (i, 0, 0)),
            pl.BlockSpec((C, C), lambda i: (0, 0)),
            pl.BlockSpec((C, C), lambda i: (0, 0)),
            pl.BlockSpec((C, C), lambda i: (0, 0)),
            pl.BlockSpec((C, C), lambda i: (0, 0)),
        ],
        out_specs=[
            pl.BlockSpec((1, N, C), lambda i: (i, 0, 0)),
            pl.BlockSpec((1, N, TD), lambda i: (i, 0, 0)),
        ],
        out_shape=[
            jax.ShapeDtypeStruct((B, N, C), F32),
            jax.ShapeDtypeStruct((B, N, TD), F32),
        ],
    )(x, pp, bp['lin1'], bp['wq'], bp['wk'], bp['wv'])


def _attn(pp, q, x, tj, bp, k, want_table):
    B, M, C = x.shape
    D = _pad128(2 * C + 16)
    TW = _pad128(16 + C)
    MT = min(M, 512)
    wp1p = jnp.pad(bp['wp1'], ((0, 13), (0, 0)))

    def body(p_ref, q_ref, x_ref, tj_ref, wp1, wp2, wa1, wa2, l2, o_ref,
             *rest):
        pt = p_ref[0]
        qt = q_ref[0]
        xt = x_ref[0]
        es, poss, vs = [], [], []
        for j in range(k):
            tjj = tj_ref[j]
            kj = tjj[:, :C]
            vj = tjj[:, C:2 * C]
            pj = tjj[:, 2 * C:2 * C + 16]
            pos = _dot(_relu(_dot(pt - pj, wp1[...])), wp2[...])
            e = _dot(_relu(_dot(qt - kj + pos, wa1[...])), wa2[...])
            es.append(e)
            poss.append(pos)
            vs.append(vj)
        m = es[0]
        for j in range(1, k):
            m = jnp.maximum(m, es[j])
        s = None
        acc = None
        for j in range(k):
            w_ = jnp.exp(es[j] - m)
            s = w_ if s is None else s + w_
            t_ = w_ * (vs[j] + poss[j])
            acc = t_ if acc is None else acc + t_
        oo = _relu(_dot(acc / s, l2[...]) + xt)
        o_ref[0] = oo
        if rest:
            rest[0][0] = jnp.concatenate(
                [pt, oo, jnp.zeros((MT, TW - 16 - C), F32)], axis=1)

    out_specs = [pl.BlockSpec((1, MT, C), lambda i, j: (i, j, 0))]
    out_shape = [jax.ShapeDtypeStruct((B, M, C), F32)]
    if want_table:
        out_specs.append(pl.BlockSpec((1, MT, TW), lambda i, j: (i, j, 0)))
        out_shape.append(jax.ShapeDtypeStruct((B, M, TW), F32))

    res = pl.pallas_call(
        body,
        grid=(B, M // MT),
        in_specs=[
            pl.BlockSpec((1, MT, 16), lambda i, j: (i, j, 0)),
            pl.BlockSpec((1, MT, C), lambda i, j: (i, j, 0)),
            pl.BlockSpec((1, MT, C), lambda i, j: (i, j, 0)),
            pl.BlockSpec((k, MT, D), lambda i, j: (i, j, 0)),
            pl.BlockSpec((16, C), lambda i, j: (0, 0)),
            pl.BlockSpec((C, C), lambda i, j: (0, 0)),
            pl.BlockSpec((C, C), lambda i, j: (0, 0)),
            pl.BlockSpec((C, C), lambda i, j: (0, 0)),
            pl.BlockSpec((C, C), lambda i, j: (0, 0)),
        ],
        out_specs=out_specs,
        out_shape=out_shape,
    )(pp, q, x, tj, wp1p, bp['wp2'], bp['wa1'], bp['wa2'], bp['lin2'])
    return res if want_table else res[0]


def _td(pn, tj, w3, wx, g, b, k):
    B, M, _ = pn.shape
    ci, co = wx.shape
    TW = tj.shape[2]
    cnt = float(k * M * co)

    def body(p_ref, tj_ref, w3r, wxr, gr, br, o_ref):
        pt = p_ref[0]
        fs = []
        tot = None
        for j in range(k):
            tjj = tj_ref[j]
            f = (_dot(pt - tjj[:, :16], w3r[...])
                 + _dot(tjj[:, 16:16 + ci], wxr[...]))
            fs.append(f)
            sj = jnp.sum(f)
            tot = sj if tot is None else tot + sj
        mu = tot / cnt
        vtot = None
        for j in range(k):
            vj = jnp.sum((fs[j] - mu) ** 2)
            vtot = vj if vtot is None else vtot + vj
        rs = jnp.sqrt(vtot / cnt + 1e-5)
        out = None
        for j in range(k):
            fn = _relu((fs[j] - mu) / rs * gr[0:1, :] + br[0:1, :])
            out = fn if out is None else jnp.maximum(out, fn)
        o_ref[0] = out

    return pl.pallas_call(
        body,
        grid=(B,),
        in_specs=[
            pl.BlockSpec((1, M, 16), lambda i: (i, 0, 0)),
            pl.BlockSpec((k, M, TW), lambda i: (i, 0, 0)),
            pl.BlockSpec((16, co), lambda i: (0, 0)),
            pl.BlockSpec((ci, co), lambda i: (0, 0)),
            pl.BlockSpec((8, co), lambda i: (0, 0)),
            pl.BlockSpec((8, co), lambda i: (0, 0)),
        ],
        out_specs=pl.BlockSpec((1, M, co), lambda i: (i, 0, 0)),
        out_shape=jax.ShapeDtypeStruct((B, M, co), F32),
    )(pn, tj, w3, wx, _row8(g), _row8(b))


def _final(x, w, b):
    B, M, C = x.shape

    def body(x_ref, w_ref, b_ref, o_ref):
        v = x_ref[...]
        m = v[:, 0, :]
        for j in range(1, M):
            m = jnp.maximum(m, v[:, j, :])
        o_ref[...] = _relu(_dot(m, w_ref[...]) + b_ref[0:1, :])

    return pl.pallas_call(
        body,
        in_specs=[
            pl.BlockSpec((B, M, C), lambda: (0, 0, 0)),
            pl.BlockSpec((C, C), lambda: (0, 0)),
            pl.BlockSpec((8, C), lambda: (0, 0)),
        ],
        out_specs=pl.BlockSpec((B, C), lambda: (0, 0)),
        out_shape=jax.ShapeDtypeStruct((B, C), F32),
    )(x, w, _row8(b))


def _block(pp, x, bp, k, want_table):
    B, M, C = x.shape
    prT = jnp.transpose(pp, (0, 2, 1))
    idx = _knn(pp, prT, k)
    q, table = _proj(x, pp, bp)
    D = _pad128(2 * C + 16)
    tj = _sc_gather(table.reshape(B * M, D), _flat_idx(idx, M))
    tj = tj.reshape(B * k, M, D)
    return _attn(pp, q, x, tj, bp, k, want_table)


def _down(pp, tdt, dp, stride, k):
    B, M, Dt = tdt.shape
    pn = pp[:, ::stride]
    Mq = M // stride
    prT = jnp.transpose(pp, (0, 2, 1))
    idx = _knn(pn, prT, k)
    tj = _sc_gather(tdt.reshape(B * M, Dt), _flat_idx(idx, M))
    tj = tj.reshape(B * k, Mq, Dt)
    w = dp['w']
    w3 = jnp.pad(w[:3], ((0, 13), (0, 0)))
    xn = _td(pn, tj, w3, w[3:], dp['g'], dp['b'], k)
    return pn, xn


def kernel(p1, x1, params):
    B, N, _ = p1.shape
    pp = jnp.pad(p1, ((0, 0), (0, 0), (0, 13)))
    x1t = jnp.pad(jnp.transpose(x1, (0, 2, 1)), ((0, 0), (0, 0), (0, 13)))
    in_w = jnp.pad(params['in_w'], ((0, 13), (0, 0)))
    x = _embed(x1t, in_w, params['in_g'], params['in_b'])

    x, tdt = _block(pp, x, params['b1'], 8, True)
    pp, x = _down(pp, tdt, params['d1'], 4, 16)
    x, tdt = _block(pp, x, params['b2'], 16, True)
    pp, x = _down(pp, tdt, params['d2'], 4, 16)
    x, tdt = _block(pp, x, params['b3'], 16, True)
    pp, x = _down(pp, tdt, params['d3'], 4, 16)
    x, tdt = _block(pp, x, params['b4'], 16, True)
    pp, x = _down(pp, tdt, params['d4'], 4, 16)
    x = _block(pp, x, params['b5'], 16, False)

    return _final(x, params['agg_w'], params['agg_b'])


# speedup vs baseline: 25.7344x; 1.1898x over previous
import functools

import jax
import jax.numpy as jnp
from jax import lax
from jax.experimental import pallas as pl
from jax.experimental.pallas import tpu as pltpu
from jax.experimental.pallas import tpu_sc as plsc

F32 = jnp.float32


def _relu(x):
    return jnp.maximum(x, 0.0)


def _pad128(n):
    return (n + 127) // 128 * 128


def _dot(a, b):
    return jnp.dot(a, b, preferred_element_type=F32)


def _row8(v):
    return jnp.broadcast_to(v[None, :], (8, v.shape[0]))


def _sc_gather(table, idx):
    R, D = table.shape
    G = idx.shape[0]
    assert G % 128 == 0 and D % 128 == 0
    nch = G // 128
    idx2 = idx.reshape(nch, 128)
    NW = 32
    nfull, rem = divmod(nch, NW)
    mesh = plsc.VectorSubcoreMesh(core_axis_name="c", subcore_axis_name="s")

    @functools.partial(
        pl.kernel,
        out_type=jax.ShapeDtypeStruct((G, D), F32),
        mesh=mesh,
        scratch_types=[
            pltpu.VMEM((128,), jnp.int32),
            pltpu.VMEM((128, D), F32),
            pltpu.SemaphoreType.DMA,
        ],
    )
    def gk(tab_hbm, idx_hbm, out_hbm, idx_v, rows_v, sem):
        wid = lax.axis_index("s") * 2 + lax.axis_index("c")
        my = nfull + jnp.where(wid < rem, 1, 0)

        def body(i, carry):
            ci = i * NW + wid
            pltpu.sync_copy(idx_hbm.at[ci], idx_v)
            pltpu.async_copy(tab_hbm.at[idx_v], rows_v, sem).wait()
            pltpu.sync_copy(rows_v, out_hbm.at[pl.ds(ci * 128, 128)])
            return carry

        lax.fori_loop(0, my, body, 0)

    return gk(table, idx2)


def _flat_idx(idx, nrows):
    B = idx.shape[0]
    perm = jnp.transpose(idx, (0, 2, 1))
    offs = (jnp.arange(B, dtype=jnp.int32) * nrows)[:, None, None]
    return (perm + offs).reshape(-1).astype(jnp.int32)


def _embed(x1t, w, g, b):
    B, N, Kp = x1t.shape
    C = w.shape[1]

    def body(x_ref, w_ref, g_ref, b_ref, o_ref):
        h = _dot(x_ref[0], w_ref[...])
        mu = jnp.mean(h)
        var = jnp.mean((h - mu) ** 2)
        hn = (h - mu) / jnp.sqrt(var + 1e-5) * g_ref[0:1, :] + b_ref[0:1, :]
        o_ref[0] = _relu(hn)

    return pl.pallas_call(
        body,
        grid=(B,),
        in_specs=[
            pl.BlockSpec((1, N, Kp), lambda i: (i, 0, 0)),
            pl.BlockSpec((Kp, C), lambda i: (0, 0)),
            pl.BlockSpec((8, C), lambda i: (0, 0)),
            pl.BlockSpec((8, C), lambda i: (0, 0)),
        ],
        out_specs=pl.BlockSpec((1, N, C), lambda i: (i, 0, 0)),
        out_shape=jax.ShapeDtypeStruct((B, N, C), F32),
    )(x1t, w, _row8(g), _row8(b))


def _knn(pq, prT, k):
    B, M, _ = pq.shape
    N = prT.shape[2]
    MT = min(M, max(8, (1 << 21) // N))

    def body(q_ref, r_ref, o_ref):
        q = q_ref[0]
        rT = r_ref[0]
        cross = _dot(q, rT)
        sqq = jnp.sum(q * q, axis=1, keepdims=True)
        sqr = jnp.sum(rT * rT, axis=0, keepdims=True)
        d = sqq - 2.0 * cross + sqr
        ii = lax.broadcasted_iota(jnp.int32, (MT, N), 1)
        key = lax.bitcast_convert_type(d, jnp.int32)
        key = (key & ~0xFFF) | ii
        cols = []
        for _ in range(k - 2):
            m = jnp.min(key, axis=1, keepdims=True)
            cols.append(m & 0xFFF)
            key = jnp.where(key == m, 0x7FFFFFFF, key)
        d = jnp.where(key == 0x7FFFFFFF, 1e30, d)
        for _ in range(2):
            m = jnp.min(d, axis=1, keepdims=True)
            cand = jnp.where(d == m, ii, N)
            sel = jnp.min(cand, axis=1, keepdims=True)
            cols.append(sel)
            d = jnp.where(ii == sel, 1e30, d)
        o_ref[0] = jnp.concatenate(cols, axis=1)

    return pl.pallas_call(
        body,
        grid=(B, M // MT),
        in_specs=[
            pl.BlockSpec((1, MT, 16), lambda i, j: (i, j, 0)),
            pl.BlockSpec((1, 16, N), lambda i, j: (i, 0, 0)),
        ],
        out_specs=pl.BlockSpec((1, MT, k), lambda i, j: (i, j, 0)),
        out_shape=jax.ShapeDtypeStruct((B, M, k), jnp.int32),
    )(pq, prT)


def _proj(x, pp, bp):
    B, N, C = x.shape
    TD = _pad128(2 * C + 16)

    def body(x_ref, p_ref, l1, wq, wk, wv, q_ref, t_ref):
        h = _dot(x_ref[0], l1[...])
        q_ref[0] = _dot(h, wq[...])
        t_ref[0] = jnp.concatenate(
            [_dot(h, wk[...]), _dot(h, wv[...]), p_ref[0],
             jnp.zeros((N, TD - 2 * C - 16), F32)], axis=1)

    return pl.pallas_call(
        body,
        grid=(B,),
        in_specs=[
            pl.BlockSpec((1, N, C), lambda i: (i, 0, 0)),
            pl.BlockSpec((1, N, 16), lambda i: (i, 0, 0)),
            pl.BlockSpec((C, C), lambda i: (0, 0)),
            pl.BlockSpec((C, C), lambda i: (0, 0)),
            pl.BlockSpec((C, C), lambda i: (0, 0)),
            pl.BlockSpec((C, C), lambda i: (0, 0)),
        ],
        out_specs=[
            pl.BlockSpec((1, N, C), lambda i: (i, 0, 0)),
            pl.BlockSpec((1, N, TD), lambda i: (i, 0, 0)),
        ],
        out_shape=[
            jax.ShapeDtypeStruct((B, N, C), F32),
            jax.ShapeDtypeStruct((B, N, TD), F32),
        ],
    )(x, pp, bp['lin1'], bp['wq'], bp['wk'], bp['wv'])


def _attn(pp, q, x, tj, bp, k, want_table):
    B, M, C = x.shape
    D = _pad128(2 * C + 16)
    TW = _pad128(16 + C)
    MT = min(M, 512)
    wp1p = jnp.pad(bp['wp1'], ((0, 13), (0, 0)))

    def body(p_ref, q_ref, x_ref, tj_ref, wp1, wp2, wa1, wa2, l2, o_ref,
             *rest):
        pt = p_ref[0]
        qt = q_ref[0]
        xt = x_ref[0]
        es, poss, vs = [], [], []
        for j in range(k):
            tjj = tj_ref[j]
            kj = tjj[:, :C]
            vj = tjj[:, C:2 * C]
            pj = tjj[:, 2 * C:2 * C + 16]
            pos = _dot(_relu(_dot(pt - pj, wp1[...])), wp2[...])
            e = _dot(_relu(_dot(qt - kj + pos, wa1[...])), wa2[...])
            es.append(e)
            poss.append(pos)
            vs.append(vj)
        m = es[0]
        for j in range(1, k):
            m = jnp.maximum(m, es[j])
        s = None
        acc = None
        for j in range(k):
            w_ = jnp.exp(es[j] - m)
            s = w_ if s is None else s + w_
            t_ = w_ * (vs[j] + poss[j])
            acc = t_ if acc is None else acc + t_
        oo = _relu(_dot(acc / s, l2[...]) + xt)
        o_ref[0] = oo
        if rest:
            rest[0][0] = jnp.concatenate(
                [pt, oo, jnp.zeros((MT, TW - 16 - C), F32)], axis=1)

    out_specs = [pl.BlockSpec((1, MT, C), lambda i, j: (i, j, 0))]
    out_shape = [jax.ShapeDtypeStruct((B, M, C), F32)]
    if want_table:
        out_specs.append(pl.BlockSpec((1, MT, TW), lambda i, j: (i, j, 0)))
        out_shape.append(jax.ShapeDtypeStruct((B, M, TW), F32))

    res = pl.pallas_call(
        body,
        grid=(B, M // MT),
        in_specs=[
            pl.BlockSpec((1, MT, 16), lambda i, j: (i, j, 0)),
            pl.BlockSpec((1, MT, C), lambda i, j: (i, j, 0)),
            pl.BlockSpec((1, MT, C), lambda i, j: (i, j, 0)),
            pl.BlockSpec((k, MT, D), lambda i, j: (i, j, 0)),
            pl.BlockSpec((16, C), lambda i, j: (0, 0)),
            pl.BlockSpec((C, C), lambda i, j: (0, 0)),
            pl.BlockSpec((C, C), lambda i, j: (0, 0)),
            pl.BlockSpec((C, C), lambda i, j: (0, 0)),
            pl.BlockSpec((C, C), lambda i, j: (0, 0)),
        ],
        out_specs=out_specs,
        out_shape=out_shape,
    )(pp, q, x, tj, wp1p, bp['wp2'], bp['wa1'], bp['wa2'], bp['lin2'])
    return res if want_table else res[0]


def _td(pn, tj, w3, wx, g, b, k):
    B, M, _ = pn.shape
    ci, co = wx.shape
    TW = tj.shape[2]
    cnt = float(k * M * co)

    def body(p_ref, tj_ref, w3r, wxr, gr, br, o_ref):
        pt = p_ref[0]
        fs = []
        tot = None
        for j in range(k):
            tjj = tj_ref[j]
            f = (_dot(pt - tjj[:, :16], w3r[...])
                 + _dot(tjj[:, 16:16 + ci], wxr[...]))
            fs.append(f)
            sj = jnp.sum(f)
            tot = sj if tot is None else tot + sj
        mu = tot / cnt
        vtot = None
        for j in range(k):
            vj = jnp.sum((fs[j] - mu) ** 2)
            vtot = vj if vtot is None else vtot + vj
        rs = jnp.sqrt(vtot / cnt + 1e-5)
        out = None
        for j in range(k):
            fn = _relu((fs[j] - mu) / rs * gr[0:1, :] + br[0:1, :])
            out = fn if out is None else jnp.maximum(out, fn)
        o_ref[0] = out

    return pl.pallas_call(
        body,
        grid=(B,),
        in_specs=[
            pl.BlockSpec((1, M, 16), lambda i: (i, 0, 0)),
            pl.BlockSpec((k, M, TW), lambda i: (i, 0, 0)),
            pl.BlockSpec((16, co), lambda i: (0, 0)),
            pl.BlockSpec((ci, co), lambda i: (0, 0)),
            pl.BlockSpec((8, co), lambda i: (0, 0)),
            pl.BlockSpec((8, co), lambda i: (0, 0)),
        ],
        out_specs=pl.BlockSpec((1, M, co), lambda i: (i, 0, 0)),
        out_shape=jax.ShapeDtypeStruct((B, M, co), F32),
    )(pn, tj, w3, wx, _row8(g), _row8(b))


def _final(x, w, b):
    B, M, C = x.shape

    def body(x_ref, w_ref, b_ref, o_ref):
        v = x_ref[...]
        m = v[:, 0, :]
        for j in range(1, M):
            m = jnp.maximum(m, v[:, j, :])
        o_ref[...] = _relu(_dot(m, w_ref[...]) + b_ref[0:1, :])

    return pl.pallas_call(
        body,
        in_specs=[
            pl.BlockSpec((B, M, C), lambda: (0, 0, 0)),
            pl.BlockSpec((C, C), lambda: (0, 0)),
            pl.BlockSpec((8, C), lambda: (0, 0)),
        ],
        out_specs=pl.BlockSpec((B, C), lambda: (0, 0)),
        out_shape=jax.ShapeDtypeStruct((B, C), F32),
    )(x, w, _row8(b))


def _block(pp, x, bp, k, want_table):
    B, M, C = x.shape
    prT = jnp.transpose(pp, (0, 2, 1))
    idx = _knn(pp, prT, k)
    q, table = _proj(x, pp, bp)
    D = _pad128(2 * C + 16)
    tj = _sc_gather(table.reshape(B * M, D), _flat_idx(idx, M))
    tj = tj.reshape(B * k, M, D)
    return _attn(pp, q, x, tj, bp, k, want_table)


def _down(pp, tdt, dp, stride, k):
    B, M, Dt = tdt.shape
    pn = pp[:, ::stride]
    Mq = M // stride
    prT = jnp.transpose(pp, (0, 2, 1))
    idx = _knn(pn, prT, k)
    tj = _sc_gather(tdt.reshape(B * M, Dt), _flat_idx(idx, M))
    tj = tj.reshape(B * k, Mq, Dt)
    w = dp['w']
    w3 = jnp.pad(w[:3], ((0, 13), (0, 0)))
    xn = _td(pn, tj, w3, w[3:], dp['g'], dp['b'], k)
    return pn, xn


def kernel(p1, x1, params):
    B, N, _ = p1.shape
    pp = jnp.pad(p1, ((0, 0), (0, 0), (0, 13)))
    x1t = jnp.pad(jnp.transpose(x1, (0, 2, 1)), ((0, 0), (0, 0), (0, 13)))
    in_w = jnp.pad(params['in_w'], ((0, 13), (0, 0)))
    x = _embed(x1t, in_w, params['in_g'], params['in_b'])

    x, tdt = _block(pp, x, params['b1'], 8, True)
    pp, x = _down(pp, tdt, params['d1'], 4, 16)
    x, tdt = _block(pp, x, params['b2'], 16, True)
    pp, x = _down(pp, tdt, params['d2'], 4, 16)
    x, tdt = _block(pp, x, params['b3'], 16, True)
    pp, x = _down(pp, tdt, params['d3'], 4, 16)
    x, tdt = _block(pp, x, params['b4'], 16, True)
    pp, x = _down(pp, tdt, params['d4'], 4, 16)
    x = _block(pp, x, params['b5'], 16, False)

    return _final(x, params['agg_w'], params['agg_b'])
